# initial kernel scaffold (unmeasured)
import jax
import jax.numpy as jnp
from jax import lax
from jax.experimental import pallas as pl
from jax.experimental.pallas import tpu as pltpu

T = 4096
D = 1024
CHUNK = 512
MAX_CHUNKS = T // CHUNK


def kernel(x, dest):
    order = jnp.argsort(dest, stable=True)
    xs = x[order].astype(jnp.bfloat16)
    n0 = jnp.sum(dest == 0).astype(jnp.int32).reshape((1,))

    def body(sc_ref, xs_ref, out_ref, send_sems, recv_sems):
        me = lax.axis_index("y")
        my_x = lax.axis_index("x")
        peer = (my_x, 1 - me)

        n0 = sc_ref[0]
        n_keep = jnp.where(me == 0, n0, T - n0)
        n_send = T - n_keep
        n_recv = T - n_keep
        keep_src = jnp.where(me == 0, 0, n0)
        send_src = jnp.where(me == 0, n0, 0)
        keep_dst = jnp.where(me == 0, 0, T - n_keep)
        send_dst = jnp.where(me == 0, 0, T - n_send)
        nc_send = (n_send + CHUNK - 1) // CHUNK
        nc_recv = (n_recv + CHUNK - 1) // CHUNK
        nc_keep = (n_keep + CHUNK - 1) // CHUNK

        barrier = pltpu.get_barrier_semaphore()
        pl.semaphore_signal(
            barrier, inc=1, device_id=peer,
            device_id_type=pl.DeviceIdType.MESH,
        )
        pl.semaphore_wait(barrier, 1)

        def chunk_off(i, n):
            return jnp.maximum(0, jnp.minimum(i * CHUNK, n - CHUNK))

        for i in range(MAX_CHUNKS):
            @pl.when(i < nc_send)
            def _(i=i):
                off = chunk_off(i, n_send)
                rdma = pltpu.make_async_remote_copy(
                    src_ref=xs_ref.at[pl.ds(send_src + off, CHUNK)],
                    dst_ref=out_ref.at[pl.ds(send_dst + off, CHUNK)],
                    send_sem=send_sems.at[i],
                    recv_sem=recv_sems.at[i],
                    device_id=peer,
                    device_id_type=pl.DeviceIdType.MESH,
                )
                rdma.start()

        for i in range(MAX_CHUNKS):
            @pl.when(i < nc_keep)
            def _(i=i):
                off = chunk_off(i, n_keep)
                out_ref[pl.ds(keep_dst + off, CHUNK), :] = (
                    xs_ref[pl.ds(keep_src + off, CHUNK), :]
                )

        for i in range(MAX_CHUNKS):
            @pl.when(i < nc_recv)
            def _(i=i):
                rdma = pltpu.make_async_remote_copy(
                    src_ref=xs_ref.at[pl.ds(0, CHUNK)],
                    dst_ref=out_ref.at[pl.ds(0, CHUNK)],
                    send_sem=send_sems.at[i],
                    recv_sem=recv_sems.at[i],
                    device_id=peer,
                    device_id_type=pl.DeviceIdType.MESH,
                )
                rdma.wait_recv()

        for i in range(MAX_CHUNKS):
            @pl.when(i < nc_send)
            def _(i=i):
                rdma = pltpu.make_async_remote_copy(
                    src_ref=xs_ref.at[pl.ds(0, CHUNK)],
                    dst_ref=out_ref.at[pl.ds(0, CHUNK)],
                    send_sem=send_sems.at[i],
                    recv_sem=recv_sems.at[i],
                    device_id=peer,
                    device_id_type=pl.DeviceIdType.MESH,
                )
                rdma.wait_send()

    return pl.pallas_call(
        body,
        out_shape=jax.ShapeDtypeStruct((T, D), jnp.bfloat16),
        in_specs=[
            pl.BlockSpec(memory_space=pltpu.SMEM),
            pl.BlockSpec(memory_space=pltpu.VMEM),
        ],
        out_specs=pl.BlockSpec(memory_space=pltpu.VMEM),
        scratch_shapes=[
            pltpu.SemaphoreType.DMA((MAX_CHUNKS,)),
            pltpu.SemaphoreType.DMA((MAX_CHUNKS,)),
        ],
        compiler_params=pltpu.CompilerParams(collective_id=0),
    )(n0, xs)


# baseline (device time: 88599 ns/iter reference)
import jax
import jax.numpy as jnp
from jax import lax
from jax.experimental import pallas as pl
from jax.experimental.pallas import tpu as pltpu

T = 4096
D = 1024
CHUNK = 512
MAX_CHUNKS = T // CHUNK


def kernel(x, dest):
    order = jnp.argsort(dest, stable=True)
    xs = x[order].astype(jnp.bfloat16).reshape(T, 8, 128)
    n0 = jnp.sum(dest == 0).astype(jnp.int32).reshape((1,))

    def body(sc_ref, xs_ref, out_ref, send_sems, recv_sems):
        me = lax.axis_index("y")
        my_x = lax.axis_index("x")
        peer = (my_x, 1 - me)

        n0 = sc_ref[0]
        n_keep = jnp.where(me == 0, n0, T - n0)
        n_send = T - n_keep
        n_recv = T - n_keep
        keep_src = jnp.where(me == 0, 0, n0)
        send_src = jnp.where(me == 0, n0, 0)
        keep_dst = jnp.where(me == 0, 0, T - n_keep)
        send_dst = jnp.where(me == 0, 0, T - n_send)
        nc_send = (n_send + CHUNK - 1) // CHUNK
        nc_recv = (n_recv + CHUNK - 1) // CHUNK
        nc_keep = (n_keep + CHUNK - 1) // CHUNK

        barrier = pltpu.get_barrier_semaphore()
        pl.semaphore_signal(
            barrier, inc=1, device_id=peer,
            device_id_type=pl.DeviceIdType.MESH,
        )
        pl.semaphore_wait(barrier, 1)

        def chunk_off(i, n):
            return jnp.maximum(0, jnp.minimum(i * CHUNK, n - CHUNK))

        for i in range(MAX_CHUNKS):
            @pl.when(i < nc_send)
            def _(i=i):
                off = chunk_off(i, n_send)
                rdma = pltpu.make_async_remote_copy(
                    src_ref=xs_ref.at[pl.ds(send_src + off, CHUNK)],
                    dst_ref=out_ref.at[pl.ds(send_dst + off, CHUNK)],
                    send_sem=send_sems.at[i],
                    recv_sem=recv_sems.at[i],
                    device_id=peer,
                    device_id_type=pl.DeviceIdType.MESH,
                )
                rdma.start()

        for i in range(MAX_CHUNKS):
            @pl.when(i < nc_keep)
            def _(i=i):
                off = chunk_off(i, n_keep)
                out_ref[pl.ds(keep_dst + off, CHUNK), :, :] = (
                    xs_ref[pl.ds(keep_src + off, CHUNK), :, :]
                )

        for i in range(MAX_CHUNKS):
            @pl.when(i < nc_recv)
            def _(i=i):
                rdma = pltpu.make_async_remote_copy(
                    src_ref=xs_ref.at[pl.ds(0, CHUNK)],
                    dst_ref=out_ref.at[pl.ds(0, CHUNK)],
                    send_sem=send_sems.at[i],
                    recv_sem=recv_sems.at[i],
                    device_id=peer,
                    device_id_type=pl.DeviceIdType.MESH,
                )
                rdma.wait_recv()

        for i in range(MAX_CHUNKS):
            @pl.when(i < nc_send)
            def _(i=i):
                rdma = pltpu.make_async_remote_copy(
                    src_ref=xs_ref.at[pl.ds(0, CHUNK)],
                    dst_ref=out_ref.at[pl.ds(0, CHUNK)],
                    send_sem=send_sems.at[i],
                    recv_sem=recv_sems.at[i],
                    device_id=peer,
                    device_id_type=pl.DeviceIdType.MESH,
                )
                rdma.wait_send()

    out = pl.pallas_call(
        body,
        out_shape=jax.ShapeDtypeStruct((T, 8, 128), jnp.bfloat16),
        in_specs=[
            pl.BlockSpec(memory_space=pltpu.SMEM),
            pl.BlockSpec(memory_space=pltpu.VMEM),
        ],
        out_specs=pl.BlockSpec(memory_space=pltpu.VMEM),
        scratch_shapes=[
            pltpu.SemaphoreType.DMA((MAX_CHUNKS,)),
            pltpu.SemaphoreType.DMA((MAX_CHUNKS,)),
        ],
        compiler_params=pltpu.CompilerParams(collective_id=0),
    )(n0, xs)
    return out.reshape(T, D)
